# Initial kernel scaffold; baseline (speedup 1.0000x reference)
import jax
import jax.numpy as jnp
from jax.experimental import pallas as pl


def kernel(x_node_feature, x_het_neighbour_list, W_node, b_node, W_neigh, b_neigh, W_het, b_het):
    raise NotImplementedError("write your pallas kernel here")


# speedup vs baseline: 23.4944x; 23.4944x over previous
import functools

import jax
import jax.numpy as jnp
from jax import lax
from jax.experimental import pallas as pl
from jax.experimental.pallas import tpu as pltpu
from jax.experimental.pallas import tpu_sc as plsc

N = 100000
K = 32
R = 3
EMBED_D = 7
OUT_D = 16
D_PAD = 16

NUM_CORES = 2
NUM_SUBCORES = 16
NW = NUM_CORES * NUM_SUBCORES
NODES_PER_W = N // NW
CHUNK_NODES = 25
CHUNKS = NODES_PER_W // CHUNK_NODES
IDX_PER_CHUNK = CHUNK_NODES * K
G_IDX = 80
G_PER_CHUNK = IDX_PER_CHUNK // G_IDX

CHUNKS_PER_R = N // CHUNK_NODES
TOTAL_IDX_CHUNKS = R * CHUNKS_PER_R

BN = 10000
GRID_N = N // BN


def _node_fc_body(x_ref, w_ref, b_ref, o_ref):
    x = x_ref[...]
    o_ref[...] = jnp.tanh(
        jnp.dot(x, w_ref[...], preferred_element_type=jnp.float32) + b_ref[...]
    )


def _final_body(agg_ref, w2_ref, b2_ref, wh_ref, bh_ref, o_ref):
    y = jnp.broadcast_to(bh_ref[...], (BN, OUT_D)).astype(jnp.float32)
    for r in range(R):
        m = agg_ref[r] * (1.0 / K)
        z = jnp.dot(m, w2_ref[r], preferred_element_type=jnp.float32) + b2_ref[r]
        z = jnp.where(z >= 0.0, z, 0.01 * z)
        y = y + jnp.dot(z, wh_ref[r], preferred_element_type=jnp.float32)
    o_ref[...] = y


def _make_sc_gather_sum(interpret=False):
    mesh = plsc.VectorSubcoreMesh(
        core_axis_name="c", subcore_axis_name="s",
        num_cores=NUM_CORES, num_subcores=NUM_SUBCORES,
    )

    @functools.partial(
        pl.kernel,
        out_type=jax.ShapeDtypeStruct((R, CHUNKS_PER_R, CHUNK_NODES, D_PAD),
                                      jnp.float32),
        mesh=mesh,
        scratch_types=[
            pltpu.VMEM((G_PER_CHUNK, G_IDX), jnp.int32),
            pltpu.VMEM((IDX_PER_CHUNK, D_PAD), jnp.float32),
            pltpu.VMEM((CHUNK_NODES, D_PAD), jnp.float32),
            pltpu.SemaphoreType.DMA,
        ],
        compiler_params=pltpu.CompilerParams(use_tc_tiling_on_sc=False),
        interpret=interpret,
    )
    def sc_gather_sum(table_hbm, idx_hbm, agg_hbm, idx_v, rows_v, out_v, sem):
        wid = lax.axis_index("s") * NUM_CORES + lax.axis_index("c")
        w_chunk_base = wid * CHUNKS

        def chunk_body(t, carry):
            r = t // CHUNKS
            c = t - r * CHUNKS
            chunk = w_chunk_base + c
            pltpu.sync_copy(idx_hbm.at[r * CHUNKS_PER_R + chunk], idx_v)
            copies = []
            for g in range(G_PER_CHUNK):
                copies.append(
                    pltpu.async_copy(
                        table_hbm.at[idx_v.at[g]],
                        rows_v.at[pl.ds(g * G_IDX, G_IDX)],
                        sem,
                    )
                )
            for cp in copies:
                cp.wait()

            def node_body(i, carry2):
                base = i * K
                acc = rows_v[base]
                for kk in range(1, K):
                    acc = acc + rows_v[base + kk]
                out_v[i] = acc
                return carry2

            lax.fori_loop(0, CHUNK_NODES, node_body, 0, unroll=False)
            pltpu.sync_copy(out_v, agg_hbm.at[r, chunk])
            return carry

        lax.fori_loop(0, R * CHUNKS, chunk_body, 0, unroll=False)

    return sc_gather_sum


def kernel(x_node_feature, x_het_neighbour_list, W_node, b_node, W_neigh,
           b_neigh, W_het, b_het):
    idx = x_het_neighbour_list.astype(jnp.int32).reshape(
        TOTAL_IDX_CHUNKS, G_PER_CHUNK, G_IDX
    )

    w_node_pad = jnp.zeros((EMBED_D, D_PAD), jnp.float32).at[:, :EMBED_D].set(W_node)
    b_node_pad = jnp.zeros((1, D_PAD), jnp.float32).at[0, :EMBED_D].set(b_node)
    w2_pad = jnp.zeros((R, D_PAD, OUT_D), jnp.float32).at[:, :EMBED_D, :].set(W_neigh)
    b2 = b_neigh.reshape(R, 1, OUT_D)
    wh = W_het.reshape(R, OUT_D, OUT_D)
    bh = b_het.reshape(1, OUT_D)

    h_pad = pl.pallas_call(
        _node_fc_body,
        grid=(GRID_N,),
        in_specs=[
            pl.BlockSpec((BN, EMBED_D), lambda i: (i, 0)),
            pl.BlockSpec((EMBED_D, D_PAD), lambda i: (0, 0)),
            pl.BlockSpec((1, D_PAD), lambda i: (0, 0)),
        ],
        out_specs=pl.BlockSpec((BN, D_PAD), lambda i: (i, 0)),
        out_shape=jax.ShapeDtypeStruct((N, D_PAD), jnp.float32),
    )(x_node_feature, w_node_pad, b_node_pad)

    agg = _make_sc_gather_sum()(h_pad, idx).reshape(R, N, D_PAD)

    out = pl.pallas_call(
        _final_body,
        grid=(GRID_N,),
        in_specs=[
            pl.BlockSpec((R, BN, D_PAD), lambda i: (0, i, 0)),
            pl.BlockSpec((R, D_PAD, OUT_D), lambda i: (0, 0, 0)),
            pl.BlockSpec((R, 1, OUT_D), lambda i: (0, 0, 0)),
            pl.BlockSpec((R, OUT_D, OUT_D), lambda i: (0, 0, 0)),
            pl.BlockSpec((1, OUT_D), lambda i: (0, 0)),
        ],
        out_specs=pl.BlockSpec((BN, OUT_D), lambda i: (i, 0)),
        out_shape=jax.ShapeDtypeStruct((N, OUT_D), jnp.float32),
    )(agg, w2_pad, b2, wh, bh)
    return out


# speedup vs baseline: 41.8008x; 1.7792x over previous
import functools

import jax
import jax.numpy as jnp
from jax import lax
from jax.experimental import pallas as pl
from jax.experimental.pallas import tpu as pltpu
from jax.experimental.pallas import tpu_sc as plsc

N = 100000
K = 32
R = 3
EMBED_D = 7
OUT_D = 16
D_PAD = 16

NUM_CORES = 2
NUM_SUBCORES = 16
NW = NUM_CORES * NUM_SUBCORES
CHUNK_NODES = 50
IDX_PER_CHUNK = CHUNK_NODES * K
G_IDX = 80
G_PER_CHUNK = IDX_PER_CHUNK // G_IDX

CHUNKS_PER_R = N // CHUNK_NODES
TOTAL_IDX_CHUNKS = R * CHUNKS_PER_R

BN = 10000
GRID_N = N // BN


def _node_fc_body(x_ref, w_ref, b_ref, o_ref):
    x = x_ref[...]
    o_ref[...] = jnp.tanh(
        jnp.dot(x, w_ref[...], preferred_element_type=jnp.float32) + b_ref[...]
    )


def _final_body(agg_ref, w2_ref, b2_ref, wh_ref, bh_ref, o_ref):
    y = jnp.broadcast_to(bh_ref[...], (BN, OUT_D)).astype(jnp.float32)
    for r in range(R):
        m = agg_ref[r] * (1.0 / K)
        z = jnp.dot(m, w2_ref[r], preferred_element_type=jnp.float32) + b2_ref[r]
        z = jnp.where(z >= 0.0, z, 0.01 * z)
        y = y + jnp.dot(z, wh_ref[r], preferred_element_type=jnp.float32)
    o_ref[...] = y


def _make_sc_gather_sum(interpret=False):
    mesh = plsc.VectorSubcoreMesh(
        core_axis_name="c", subcore_axis_name="s",
        num_cores=NUM_CORES, num_subcores=NUM_SUBCORES,
    )

    @functools.partial(
        pl.kernel,
        out_type=jax.ShapeDtypeStruct((R, CHUNKS_PER_R, CHUNK_NODES, D_PAD),
                                      jnp.float32),
        mesh=mesh,
        scratch_types=[
            pltpu.VMEM((2, G_PER_CHUNK, G_IDX), jnp.int32),
            pltpu.VMEM((2, IDX_PER_CHUNK, D_PAD), jnp.float32),
            pltpu.VMEM((2, CHUNK_NODES, D_PAD), jnp.float32),
            pltpu.SemaphoreType.DMA,
            pltpu.SemaphoreType.DMA,
            pltpu.SemaphoreType.DMA,
        ],
        compiler_params=pltpu.CompilerParams(use_tc_tiling_on_sc=False),
        interpret=interpret,
    )
    def sc_gather_sum(table_hbm, idx_hbm, agg_hbm, idx_v, rows_v, out_v,
                      sem_i, sem_g, sem_o):
        wid = lax.axis_index("s") * NUM_CORES + lax.axis_index("c")
        n_items = (TOTAL_IDX_CHUNKS - wid + NW - 1) // NW
        last_chunk = TOTAL_IDX_CHUNKS - 1

        def fire_gathers(slot):
            for g in range(G_PER_CHUNK):
                pltpu.async_copy(
                    table_hbm.at[idx_v.at[slot, g]],
                    rows_v.at[slot, pl.ds(g * G_IDX, G_IDX)],
                    sem_g,
                )

        def wait_gathers(slot):
            for g in range(G_PER_CHUNK):
                pltpu.make_async_copy(
                    table_hbm.at[pl.ds(0, G_IDX)],
                    rows_v.at[slot, pl.ds(g * G_IDX, G_IDX)],
                    sem_g,
                ).wait()

        pltpu.async_copy(idx_hbm.at[wid], idx_v.at[0], sem_i).wait()
        fire_gathers(0)
        pltpu.async_copy(
            idx_hbm.at[jnp.minimum(wid + NW, last_chunk)], idx_v.at[1], sem_i
        )

        def item_body(i, carry):
            slot = lax.rem(i, 2)
            nslot = 1 - slot
            gch = wid + i * NW
            wait_gathers(slot)
            pltpu.make_async_copy(
                idx_hbm.at[0], idx_v.at[nslot], sem_i
            ).wait()
            pltpu.async_copy(
                idx_hbm.at[jnp.minimum(gch + 2 * NW, last_chunk)],
                idx_v.at[slot], sem_i,
            )
            fire_gathers(nslot)

            def node_body(nd, carry2):
                base = nd * K
                acc = rows_v[slot, base]
                for kk in range(1, K):
                    acc = acc + rows_v[slot, base + kk]
                out_v[slot, nd] = acc
                return carry2

            lax.fori_loop(0, CHUNK_NODES, node_body, 0, unroll=False)

            @pl.when(i > 0)
            def _():
                pltpu.make_async_copy(
                    out_v.at[nslot], agg_hbm.at[0, 0], sem_o
                ).wait()

            r_i = gch // CHUNKS_PER_R
            c_i = gch - r_i * CHUNKS_PER_R
            pltpu.async_copy(out_v.at[slot], agg_hbm.at[r_i, c_i], sem_o)
            return carry

        lax.fori_loop(0, n_items, item_body, 0, unroll=False)

        last_slot = lax.rem(n_items, 2)
        wait_gathers(last_slot)
        pltpu.make_async_copy(idx_hbm.at[0], idx_v.at[0], sem_i).wait()
        pltpu.make_async_copy(
            out_v.at[1 - last_slot], agg_hbm.at[0, 0], sem_o
        ).wait()

    return sc_gather_sum


def kernel(x_node_feature, x_het_neighbour_list, W_node, b_node, W_neigh,
           b_neigh, W_het, b_het):
    idx = x_het_neighbour_list.astype(jnp.int32).reshape(
        TOTAL_IDX_CHUNKS, G_PER_CHUNK, G_IDX
    )

    w_node_pad = jnp.zeros((EMBED_D, D_PAD), jnp.float32).at[:, :EMBED_D].set(W_node)
    b_node_pad = jnp.zeros((1, D_PAD), jnp.float32).at[0, :EMBED_D].set(b_node)
    w2_pad = jnp.zeros((R, D_PAD, OUT_D), jnp.float32).at[:, :EMBED_D, :].set(W_neigh)
    b2 = b_neigh.reshape(R, 1, OUT_D)
    wh = W_het.reshape(R, OUT_D, OUT_D)
    bh = b_het.reshape(1, OUT_D)

    h_pad = pl.pallas_call(
        _node_fc_body,
        grid=(GRID_N,),
        in_specs=[
            pl.BlockSpec((BN, EMBED_D), lambda i: (i, 0)),
            pl.BlockSpec((EMBED_D, D_PAD), lambda i: (0, 0)),
            pl.BlockSpec((1, D_PAD), lambda i: (0, 0)),
        ],
        out_specs=pl.BlockSpec((BN, D_PAD), lambda i: (i, 0)),
        out_shape=jax.ShapeDtypeStruct((N, D_PAD), jnp.float32),
    )(x_node_feature, w_node_pad, b_node_pad)

    agg = _make_sc_gather_sum()(h_pad, idx).reshape(R, N, D_PAD)

    out = pl.pallas_call(
        _final_body,
        grid=(GRID_N,),
        in_specs=[
            pl.BlockSpec((R, BN, D_PAD), lambda i: (0, i, 0)),
            pl.BlockSpec((R, D_PAD, OUT_D), lambda i: (0, 0, 0)),
            pl.BlockSpec((R, 1, OUT_D), lambda i: (0, 0, 0)),
            pl.BlockSpec((R, OUT_D, OUT_D), lambda i: (0, 0, 0)),
            pl.BlockSpec((1, OUT_D), lambda i: (0, 0)),
        ],
        out_specs=pl.BlockSpec((BN, OUT_D), lambda i: (i, 0)),
        out_shape=jax.ShapeDtypeStruct((N, OUT_D), jnp.float32),
    )(agg, w2_pad, b2, wh, bh)
    return out
